# baseline (speedup 1.0000x reference)
import functools

import jax
import jax.numpy as jnp
from jax import lax
from jax.experimental import pallas as pl
from jax.experimental.pallas import tpu as pltpu
from jax.experimental.pallas import tpu_sc as plsc

NE = 1000000
NR = 1000
D = 32
B = 16384
L = 16
CHUNK = 128


def _make_sc_call():
    info = plsc.get_sparse_core_info()
    nc, ns = info.num_cores, info.num_subcores
    nw = nc * ns
    bpw = B // nw
    nchunks = bpw // CHUNK

    mesh = plsc.VectorSubcoreMesh(core_axis_name="c", subcore_axis_name="s")

    @functools.partial(
        pl.kernel,
        mesh=mesh,
        out_type=jax.ShapeDtypeStruct((nw, L), jnp.float32),
        compiler_params=pltpu.CompilerParams(
            needs_layout_passes=False, use_tc_tiling_on_sc=False),
        scratch_types=[
            pltpu.VMEM((bpw,), jnp.int32),
            pltpu.VMEM((bpw,), jnp.int32),
            pltpu.VMEM((bpw,), jnp.int32),
            pltpu.VMEM((bpw,), jnp.int32),
            pltpu.VMEM((bpw,), jnp.int32),
            pltpu.VMEM((bpw, D), jnp.float32),
            pltpu.VMEM((bpw, D), jnp.float32),
            pltpu.VMEM((bpw, D), jnp.float32),
            pltpu.VMEM((bpw, D), jnp.float32),
            pltpu.VMEM((bpw, D), jnp.float32),
            pltpu.VMEM((L,), jnp.float32),
            pltpu.SemaphoreType.DMA,
        ],
    )
    def trans_e(ph_hbm, pr_hbm, pt_hbm, nh_hbm, nr_hbm, ent_hbm, rel_hbm,
                out_hbm,
                ph_i, pr_i, pt_i, nh_i, nr_i,
                ph_v, pr_v, pt_v, nh_v, nr_v,
                acc_v, sem):
        wid = lax.axis_index("s") * nc + lax.axis_index("c")
        base = wid * bpw

        pltpu.sync_copy(ph_hbm.at[pl.ds(base, bpw)], ph_i)
        pltpu.sync_copy(pr_hbm.at[pl.ds(base, bpw)], pr_i)
        pltpu.sync_copy(pt_hbm.at[pl.ds(base, bpw)], pt_i)
        pltpu.sync_copy(nh_hbm.at[pl.ds(base, bpw)], nh_i)
        pltpu.sync_copy(nr_hbm.at[pl.ds(base, bpw)], nr_i)

        copies = []
        for j in range(nchunks):
            sl = pl.ds(j * CHUNK, CHUNK)
            for tbl, idx, dst in ((ent_hbm, ph_i, ph_v),
                                  (rel_hbm, pr_i, pr_v),
                                  (ent_hbm, pt_i, pt_v),
                                  (ent_hbm, nh_i, nh_v),
                                  (rel_hbm, nr_i, nr_v)):
                copies.append(
                    pltpu.async_copy(tbl.at[idx.at[sl]], dst.at[sl], sem))
        for c in copies:
            c.wait()

        zeros = jnp.zeros((L,), jnp.float32)
        iota = lax.iota(jnp.int32, L)

        def body(g, acc):
            row_idx = iota + g * L
            dpos = zeros
            dneg = zeros
            for d in range(D):
                col = jnp.full((L,), d, jnp.int32)
                ph_c = plsc.load_gather(ph_v, [row_idx, col])
                pr_c = plsc.load_gather(pr_v, [row_idx, col])
                pt_c = plsc.load_gather(pt_v, [row_idx, col])
                nh_c = plsc.load_gather(nh_v, [row_idx, col])
                nr_c = plsc.load_gather(nr_v, [row_idx, col])
                dpos = dpos + jnp.abs(ph_c + pr_c - pt_c)
                dneg = dneg + jnp.abs(nh_c + nr_c - pt_c)
            return acc + jnp.maximum(dpos - dneg + 1.0, 0.0)

        acc = lax.fori_loop(0, bpw // L, body, zeros)
        acc_v[...] = acc
        pltpu.sync_copy(acc_v, out_hbm.at[wid])

    return trans_e


def kernel(pos_h, pos_r, pos_t, neg_h, neg_r, neg_t, entity_embds, rel_embds):
    del neg_t
    call = _make_sc_call()
    partials = call(pos_h.astype(jnp.int32), pos_r.astype(jnp.int32),
                    pos_t.astype(jnp.int32), neg_h.astype(jnp.int32),
                    neg_r.astype(jnp.int32), entity_embds, rel_embds)
    return jnp.sum(partials) * (1.0 / B)


# speedup vs baseline: 1.0120x; 1.0120x over previous
import functools

import jax
import jax.numpy as jnp
from jax import lax
from jax.experimental import pallas as pl
from jax.experimental.pallas import tpu as pltpu
from jax.experimental.pallas import tpu_sc as plsc

NE = 1000000
NR = 1000
D = 32
B = 16384
L = 16
W = 128
RPW = W // D
CH = 64
NBUF = 2


def _make_sc_call():
    info = plsc.get_sparse_core_info()
    nc, ns = info.num_cores, info.num_subcores
    nw = nc * ns
    bpw = B // nw
    nch = bpw // CH

    mesh = plsc.VectorSubcoreMesh(core_axis_name="c", subcore_axis_name="s")

    @functools.partial(
        pl.kernel,
        mesh=mesh,
        out_type=jax.ShapeDtypeStruct((nw, L), jnp.float32),
        compiler_params=pltpu.CompilerParams(needs_layout_passes=False),
        scratch_types=[
            pltpu.VMEM((bpw,), jnp.int32),
            pltpu.VMEM((bpw,), jnp.int32),
            pltpu.VMEM((bpw,), jnp.int32),
            pltpu.VMEM((bpw,), jnp.int32),
            pltpu.VMEM((bpw,), jnp.int32),
            pltpu.VMEM((bpw,), jnp.int32),
            pltpu.VMEM((bpw,), jnp.int32),
            pltpu.VMEM((bpw,), jnp.int32),
            pltpu.VMEM((NBUF, CH, W), jnp.float32),
            pltpu.VMEM((NBUF, CH, W), jnp.float32),
            pltpu.VMEM((NBUF, CH, W), jnp.float32),
            pltpu.VMEM((NR // RPW, W), jnp.float32),
            pltpu.VMEM((L,), jnp.float32),
            pltpu.SemaphoreType.DMA,
            pltpu.SemaphoreType.DMA,
        ],
    )
    def trans_e(ph_hbm, pr_hbm, pt_hbm, nh_hbm, nr_hbm, ent_hbm, rel_hbm,
                out_hbm,
                ph_i, pr_i, pt_i, nh_i, nr_i,
                ph_t, pt_t, nh_t,
                ph_v, pt_v, nh_v,
                rel_v, acc_v, sem0, sem1):
        wid = lax.axis_index("s") * nc + lax.axis_index("c")
        base = wid * bpw
        sems = (sem0, sem1)

        rel_copy = pltpu.async_copy(rel_hbm, rel_v, sem0)

        pltpu.sync_copy(ph_hbm.at[pl.ds(base, bpw)], ph_i)
        pltpu.sync_copy(pr_hbm.at[pl.ds(base, bpw)], pr_i)
        pltpu.sync_copy(pt_hbm.at[pl.ds(base, bpw)], pt_i)
        pltpu.sync_copy(nh_hbm.at[pl.ds(base, bpw)], nh_i)
        pltpu.sync_copy(nr_hbm.at[pl.ds(base, bpw)], nr_i)

        def shift_body(k, _):
            sl = pl.ds(k * L, L)
            ph_t[sl] = ph_i[sl] >> 2
            pt_t[sl] = pt_i[sl] >> 2
            nh_t[sl] = nh_i[sl] >> 2
            return 0
        lax.fori_loop(0, bpw // L, shift_body, 0)
        rel_copy.wait()

        def fire(j, b):
            sl = pl.ds(j * CH, CH)
            pltpu.async_copy(ent_hbm.at[ph_t.at[sl]], ph_v.at[b], sems[b])
            pltpu.async_copy(ent_hbm.at[pt_t.at[sl]], pt_v.at[b], sems[b])
            pltpu.async_copy(ent_hbm.at[nh_t.at[sl]], nh_v.at[b], sems[b])

        def drain(b):
            for buf in (ph_v, pt_v, nh_v):
                pltpu.make_async_copy(
                    ent_hbm.at[pl.ds(0, CH)], buf.at[b], sems[b]).wait()

        fire(0, 0)
        fire(1, 1)

        iota = lax.iota(jnp.int32, L)
        zeros = jnp.zeros((L,), jnp.float32)
        three = jnp.full((L,), 3, jnp.int32)

        def compute_chunk(j, b, acc):
            bv = jnp.full((L,), b, jnp.int32)

            def group(g, acc):
                pos0 = j * CH + g * L
                sl = pl.ds(pos0, L)
                rowv = iota + g * L
                cb_ph = (ph_i[sl] & three) << 5
                cb_pt = (pt_i[sl] & three) << 5
                cb_nh = (nh_i[sl] & three) << 5
                pr = pr_i[sl]
                nr = nr_i[sl]
                row_pr = pr >> 2
                col_pr = (pr & three) << 5
                row_nr = nr >> 2
                col_nr = (nr & three) << 5
                dpos = zeros
                dneg = zeros
                for d in range(D):
                    phc = plsc.load_gather(ph_v, [bv, rowv, cb_ph + d])
                    ptc = plsc.load_gather(pt_v, [bv, rowv, cb_pt + d])
                    nhc = plsc.load_gather(nh_v, [bv, rowv, cb_nh + d])
                    prc = plsc.load_gather(rel_v, [row_pr, col_pr + d])
                    nrc = plsc.load_gather(rel_v, [row_nr, col_nr + d])
                    dpos = dpos + jnp.abs(phc + prc - ptc)
                    dneg = dneg + jnp.abs(nhc + nrc - ptc)
                return acc + jnp.maximum(dpos - dneg + 1.0, 0.0)

            return lax.fori_loop(0, CH // L, group, acc)

        def pair(p, acc):
            for b in range(NBUF):
                j = p * NBUF + b
                drain(b)
                acc = compute_chunk(j, b, acc)

                @pl.when(j + NBUF < nch)
                def _():
                    fire(j + NBUF, b)
            return acc

        acc = lax.fori_loop(0, nch // NBUF, pair, zeros)
        acc_v[...] = acc
        pltpu.sync_copy(acc_v, out_hbm.at[wid])

    return trans_e


def kernel(pos_h, pos_r, pos_t, neg_h, neg_r, neg_t, entity_embds, rel_embds):
    del neg_t
    call = _make_sc_call()
    ent_wide = jnp.reshape(entity_embds, (NE // RPW, W))
    rel_wide = jnp.reshape(rel_embds, (NR // RPW, W))
    partials = call(pos_h.astype(jnp.int32), pos_r.astype(jnp.int32),
                    pos_t.astype(jnp.int32), neg_h.astype(jnp.int32),
                    neg_r.astype(jnp.int32), ent_wide, rel_wide)
    return jnp.sum(partials) * (1.0 / B)
